# baseline (speedup 1.0000x reference)
import math

import jax
import jax.numpy as jnp
import numpy as np
from jax import lax
from jax.experimental import pallas as pl
from jax.experimental.pallas import tpu as pltpu
from jax.experimental.pallas import tpu_sc as plsc

VOCAB = 1000000
SIZE = 64
MAX_SEQ_LEN = 1000
BATCH = 4096
SEQ = 200

NUM_CORES = 2
NUM_SUBCORES = 16
NUM_WORKERS = NUM_CORES * NUM_SUBCORES

WIN = 256
FULL_WINS = VOCAB // WIN
REM = VOCAB - FULL_WINS * WIN
TOT_WINS = FULL_WINS + 1
MAX_K = (TOT_WINS + NUM_WORKERS - 1) // NUM_WORKERS

SCALE = math.sqrt(SIZE)


def _make_pe():
    pe = np.zeros((MAX_SEQ_LEN, SIZE), dtype=np.float32)
    position = np.arange(0, MAX_SEQ_LEN, dtype=np.float32)[:, None]
    div_term = np.exp(
        np.arange(0, SIZE, 2, dtype=np.float32) * -(math.log(10000.0) / SIZE))
    pe[:, 0::2] = np.sin(position * div_term)
    pe[:, 1::2] = np.cos(position * div_term)
    return pe[:SEQ]


_PE = _make_pe()


def _detile_body(tabT_hbm, scr_hbm, src0, src1, dst0, dst1, src_r, dst_r,
                 ls0, ls1, os0, os1, rs):
    wid = lax.axis_index("s") * NUM_CORES + lax.axis_index("c")
    iota = lax.iota(jnp.int32, 16)
    ih = lax.shift_right_logical(iota, 1)
    ip = lax.rem(iota, 2) * SIZE

    srcs = (src0, src1)
    dsts = (dst0, dst1)
    lsems = (ls0, ls1)
    osems = (os0, os1)

    def win_id(k):
        return k * NUM_WORKERS + wid

    def load(k, b):
        w = win_id(k)

        @pl.when(w < FULL_WINS)
        def _():
            pltpu.make_async_copy(
                tabT_hbm.at[:, pl.ds(w * WIN, WIN)], srcs[b], lsems[b]).start()

    def load_wait(k, b):
        w = win_id(k)

        @pl.when(w < FULL_WINS)
        def _():
            pltpu.make_async_copy(
                tabT_hbm.at[:, pl.ds(w * WIN, WIN)], srcs[b], lsems[b]).wait()

    def store(k, b):
        w = win_id(k)

        @pl.when(w < FULL_WINS)
        def _():
            pltpu.make_async_copy(
                dsts[b], scr_hbm.at[pl.ds(w * (WIN // 2), WIN // 2)],
                osems[b]).start()

    def store_wait(k, b):
        w = win_id(k)

        @pl.when(w < FULL_WINS)
        def _():
            pltpu.make_async_copy(
                dsts[b], scr_hbm.at[pl.ds(w * (WIN // 2), WIN // 2)],
                osems[b]).wait()

    def transpose(src, dst, ng):
        def d_body(d, _):
            col = ip + d

            def g_body(g, _):
                val = src[d, pl.ds(g * 16, 16)]
                row = ih + g * 8
                plsc.store_scatter(dst, [row, col], val)
                return 0
            lax.fori_loop(0, ng, g_body, 0)
            return 0
        lax.fori_loop(0, SIZE, d_body, 0)

    load(0, 0)
    load(1, 1)

    def step(k, _):
        for bb in range(2):
            @pl.when(lax.rem(k, 2) == bb)
            def _():
                @pl.when(win_id(k) < FULL_WINS)
                def _():
                    load_wait(k, bb)

                    @pl.when(k >= 2)
                    def _():
                        store_wait(k - 2, bb)

                    transpose(srcs[bb], dsts[bb], WIN // 16)
                    store(k, bb)
                load(k + 2, bb)
        return 0

    lax.fori_loop(0, MAX_K, step, 0)
    store_wait(MAX_K - 2, (MAX_K - 2) % 2)
    store_wait(MAX_K - 1, (MAX_K - 1) % 2)

    @pl.when(wid == 0)
    def _():
        pltpu.make_async_copy(
            tabT_hbm.at[:, pl.ds(FULL_WINS * WIN, REM)], src_r, rs).start()
        pltpu.make_async_copy(
            tabT_hbm.at[:, pl.ds(FULL_WINS * WIN, REM)], src_r, rs).wait()
        transpose(src_r, dst_r, REM // 16)
        pltpu.make_async_copy(
            dst_r, scr_hbm.at[pl.ds(FULL_WINS * (WIN // 2), REM // 2)],
            rs).start()
        pltpu.make_async_copy(
            dst_r, scr_hbm.at[pl.ds(FULL_WINS * (WIN // 2), REM // 2)],
            rs).wait()


def _lookup_body(scr_hbm, xT_hbm, pe_hbm, out_hbm,
                 x_v, pe_v, idx0, idx1, g0, g1, o0, o1,
                 gs0, gs1, ws0, ws1):
    wid = lax.axis_index("s") * NUM_CORES + lax.axis_index("c")
    b0 = wid * 128
    pltpu.sync_copy(xT_hbm.at[:, pl.ds(b0, 128)], x_v)
    pltpu.sync_copy(pe_hbm, pe_v)
    iota = lax.iota(jnp.int32, 16)

    idxs = (idx0, idx1)
    gbufs = (g0, g1)
    obufs = (o0, o1)
    gsems = (gs0, gs1)
    wsems = (ws0, ws1)

    def prep_and_fire(l, b):
        def g_body(g, _):
            xv = x_v[l, pl.ds(g * 16, 16)]
            idxs[b][pl.ds(g * 16, 16)] = lax.shift_right_logical(xv, 1)
            return 0
        lax.fori_loop(0, 8, g_body, 0, unroll=2)
        pltpu.make_async_copy(scr_hbm.at[idxs[b]], gbufs[b], gsems[b]).start()

    def gather_wait(b):
        pltpu.make_async_copy(scr_hbm.at[idxs[b]], gbufs[b], gsems[b]).wait()

    def out_store(l, b):
        pltpu.make_async_copy(
            obufs[b], out_hbm.at[l, :, pl.ds(b0, 128)], wsems[b]).start()

    def out_wait(l, b):
        pltpu.make_async_copy(
            obufs[b], out_hbm.at[l, :, pl.ds(b0, 128)], wsems[b]).wait()

    prep_and_fire(0, 0)
    prep_and_fire(1, 1)

    def step(l, _):
        for bb in range(2):
            @pl.when(lax.rem(l, 2) == bb)
            def _():
                gather_wait(bb)

                @pl.when(l >= 2)
                def _():
                    out_wait(l - 2, bb)

                def g16(g, _):
                    xv = x_v[l, pl.ds(g * 16, 16)]
                    pv = lax.rem(xv, 2) * SIZE
                    for j in range(16):
                        po = pv[j]
                        r = g * 16 + j
                        for c in range(4):
                            val = gbufs[bb][r, pl.ds(po + c * 16, 16)]
                            plsc.store_scatter(
                                obufs[bb],
                                [iota + c * 16, lax.broadcast(r, (16,))],
                                val * SCALE + pe_v[l, pl.ds(c * 16, 16)])
                    return 0
                lax.fori_loop(0, 8, g16, 0)
                out_store(l, bb)

                @pl.when(l + 2 < SEQ)
                def _():
                    prep_and_fire(l + 2, bb)
        return 0

    lax.fori_loop(0, SEQ, step, 0)
    out_wait(SEQ - 2, 0)
    out_wait(SEQ - 1, 1)


@jax.jit
def kernel(x, emb_table):
    mesh = plsc.VectorSubcoreMesh(core_axis_name="c", subcore_axis_name="s")
    tabT = emb_table.T
    xT = x.T.astype(jnp.int32)
    pe = jnp.asarray(_PE)

    scr = pl.kernel(
        _detile_body,
        out_type=jax.ShapeDtypeStruct((VOCAB // 2, 128), jnp.float32),
        mesh=mesh,
        compiler_params=pltpu.CompilerParams(needs_layout_passes=False),
        scratch_types=[
            pltpu.VMEM((SIZE, WIN), jnp.float32),
            pltpu.VMEM((SIZE, WIN), jnp.float32),
            pltpu.VMEM((WIN // 2, 128), jnp.float32),
            pltpu.VMEM((WIN // 2, 128), jnp.float32),
            pltpu.VMEM((SIZE, REM), jnp.float32),
            pltpu.VMEM((REM // 2, 128), jnp.float32),
            pltpu.SemaphoreType.DMA,
            pltpu.SemaphoreType.DMA,
            pltpu.SemaphoreType.DMA,
            pltpu.SemaphoreType.DMA,
            pltpu.SemaphoreType.DMA,
        ],
    )(tabT)

    outT = pl.kernel(
        _lookup_body,
        out_type=jax.ShapeDtypeStruct((SEQ, SIZE, BATCH), jnp.float32),
        mesh=mesh,
        compiler_params=pltpu.CompilerParams(needs_layout_passes=False),
        scratch_types=[
            pltpu.VMEM((SEQ, 128), jnp.int32),
            pltpu.VMEM((SEQ, SIZE), jnp.float32),
            pltpu.VMEM((128,), jnp.int32),
            pltpu.VMEM((128,), jnp.int32),
            pltpu.VMEM((128, 128), jnp.float32),
            pltpu.VMEM((128, 128), jnp.float32),
            pltpu.VMEM((SIZE, 128), jnp.float32),
            pltpu.VMEM((SIZE, 128), jnp.float32),
            pltpu.SemaphoreType.DMA,
            pltpu.SemaphoreType.DMA,
            pltpu.SemaphoreType.DMA,
            pltpu.SemaphoreType.DMA,
        ],
    )(scr, xT, pe)

    return outT.transpose(2, 0, 1)
